# baseline (speedup 1.0000x reference)
import functools

import jax
import jax.numpy as jnp
from jax import lax
from jax.experimental import pallas as pl
from jax.experimental.pallas import tpu as pltpu
from jax.experimental.pallas import tpu_sc as plsc

NUM_STYLES = 100000
EMBED_DIM = 64
BATCH = 16384

_info = plsc.get_sparse_core_info()
_NC, _NS = _info.num_cores, _info.num_subcores
_NW = _NC * _NS
_B_PER_W = BATCH // _NW


def _gather_body(weight_hbm, idx_hbm, out_hbm, idx_v, rows_v, sem):
    wid = lax.axis_index("s") * _NC + lax.axis_index("c")
    base = wid * _B_PER_W
    pltpu.sync_copy(idx_hbm.at[pl.ds(base, _B_PER_W)], idx_v)
    pltpu.async_copy(weight_hbm.at[idx_v], rows_v, sem).wait()
    pltpu.sync_copy(rows_v, out_hbm.at[pl.ds(base, _B_PER_W)])


_gather = pl.kernel(
    _gather_body,
    mesh=plsc.VectorSubcoreMesh(core_axis_name="c", subcore_axis_name="s"),
    out_type=jax.ShapeDtypeStruct((BATCH, EMBED_DIM), jnp.float32),
    scratch_types=[
        pltpu.VMEM((_B_PER_W,), jnp.int32),
        pltpu.VMEM((_B_PER_W, EMBED_DIM), jnp.float32),
        pltpu.SemaphoreType.DMA,
    ],
    compiler_params=pltpu.CompilerParams(use_tc_tiling_on_sc=False),
)


@jax.jit
def kernel(style_idx, weight):
    return _gather(weight, style_idx.astype(jnp.int32))


# speedup vs baseline: 1.4816x; 1.4816x over previous
import jax
import jax.numpy as jnp
from jax import lax
from jax.experimental import pallas as pl
from jax.experimental.pallas import tpu as pltpu
from jax.experimental.pallas import tpu_sc as plsc

NUM_STYLES = 100000
EMBED_DIM = 64
BATCH = 16384

_info = plsc.get_sparse_core_info()
_NC, _NS = _info.num_cores, _info.num_subcores
_NW = _NC * _NS
_BPW = BATCH // _NW


def _gather_body(w_hbm, idx_hbm, out_hbm, idx_v, rows_v, sem):
    wid = lax.axis_index("s") * _NC + lax.axis_index("c")
    base = wid * _BPW
    pltpu.sync_copy(idx_hbm.at[pl.ds(base, _BPW)], idx_v)

    def group_body(k, carry):
        g0 = k * 16
        vg = idx_v[pl.ds(g0, 16)]
        for t in range(16):
            r = vg[t]
            pltpu.async_copy(w_hbm.at[r], rows_v.at[g0 + t], sem)
        return carry

    lax.fori_loop(0, _BPW // 16, group_body, 0)
    pltpu.make_async_copy(out_hbm.at[pl.ds(base, _BPW)], rows_v, sem).wait()
    pltpu.sync_copy(rows_v, out_hbm.at[pl.ds(base, _BPW)])


_gather = pl.kernel(
    _gather_body,
    mesh=plsc.VectorSubcoreMesh(core_axis_name="c", subcore_axis_name="s"),
    out_type=jax.ShapeDtypeStruct((BATCH, EMBED_DIM), jnp.float32),
    scratch_types=[
        pltpu.VMEM((_BPW,), jnp.int32),
        pltpu.VMEM((_BPW, EMBED_DIM), jnp.float32),
        pltpu.SemaphoreType.DMA,
    ],
)


@jax.jit
def kernel(style_idx, weight):
    return _gather(weight, style_idx.astype(jnp.int32))


# speedup vs baseline: 1.7490x; 1.1805x over previous
import jax
import jax.numpy as jnp
from jax import lax
from jax.experimental import pallas as pl
from jax.experimental.pallas import tpu as pltpu
from jax.experimental.pallas import tpu_sc as plsc

NUM_STYLES = 100000
EMBED_DIM = 64
BATCH = 16384

_info = plsc.get_sparse_core_info()
_NC, _NS = _info.num_cores, _info.num_subcores
_NW = _NC * _NS
_BPW = BATCH // _NW


def _gather_body(w3_hbm, idx_hbm, out_hbm, idx_v, rows_v, sem):
    wid = lax.axis_index("s") * _NC + lax.axis_index("c")
    base = wid * _BPW
    pltpu.sync_copy(idx_hbm.at[pl.ds(base, _BPW)], idx_v)

    def group_body(k, carry):
        g0 = k * 16
        vg = idx_v[pl.ds(g0, 16)]
        bv = lax.shift_right_logical(vg, 3)
        rv = vg & 7
        for t in range(16):
            pltpu.async_copy(
                w3_hbm.at[bv[t], rv[t]], rows_v.at[g0 + t], sem
            )
        return carry

    lax.fori_loop(0, _BPW // 16, group_body, 0)
    pltpu.make_async_copy(out_hbm.at[pl.ds(base, _BPW)], rows_v, sem).wait()
    pltpu.sync_copy(rows_v, out_hbm.at[pl.ds(base, _BPW)])


_gather = pl.kernel(
    _gather_body,
    mesh=plsc.VectorSubcoreMesh(core_axis_name="c", subcore_axis_name="s"),
    out_type=jax.ShapeDtypeStruct((BATCH, EMBED_DIM), jnp.float32),
    scratch_types=[
        pltpu.VMEM((_BPW,), jnp.int32),
        pltpu.VMEM((_BPW, EMBED_DIM), jnp.float32),
        pltpu.SemaphoreType.DMA,
    ],
)


@jax.jit
def kernel(style_idx, weight):
    w3 = weight.reshape(NUM_STYLES // 8, 8, EMBED_DIM)
    return _gather(w3, style_idx.astype(jnp.int32))
